# baseline (speedup 1.0000x reference)
import functools

import jax
import jax.numpy as jnp
from jax import lax
from jax.experimental import pallas as pl
from jax.experimental.pallas import tpu as pltpu
from jax.experimental.pallas import tpu_sc as plsc

E = 160000
D = 256
NC = 2
NS = 16
NW = NC * NS
EP = 163840
EW = EP // NW
C = 64
NCHUNK = EW // C
NBUF = 2
UNROLL = 8

_mesh = plsc.VectorSubcoreMesh(core_axis_name="c", subcore_axis_name="s")


@functools.partial(
    pl.kernel,
    mesh=_mesh,
    compiler_params=pltpu.CompilerParams(use_tc_tiling_on_sc=False,
                                         needs_layout_passes=False),
    out_type=jax.ShapeDtypeStruct((NW, NCHUNK, C), jnp.float32),
    scratch_types=[
        pltpu.VMEM((NCHUNK, C), jnp.int32),
        pltpu.VMEM((NCHUNK, C), jnp.int32),
        pltpu.VMEM((NBUF, C, D), jnp.float32),
        pltpu.VMEM((NBUF, C, D), jnp.float32),
        pltpu.VMEM((NCHUNK, C), jnp.float32),
        pltpu.SemaphoreType.DMA,
        pltpu.SemaphoreType.DMA,
    ],
)
def _dot_edges(hu, ht, src_hbm, dst_hbm, out_hbm,
               src_v, dst_v, u_b, t_b, out_v, sem0, sem1):
    wid = lax.axis_index("s") * NC + lax.axis_index("c")
    sems = (sem0, sem1)

    pltpu.sync_copy(src_hbm.at[wid], src_v)
    pltpu.sync_copy(dst_hbm.at[wid], dst_v)

    def fire(g, b):
        pltpu.async_copy(hu.at[src_v.at[g]], u_b.at[b], sems[b])
        pltpu.async_copy(ht.at[dst_v.at[g]], t_b.at[b], sems[b])

    def wait(b):
        pltpu.make_async_copy(hu.at[pl.ds(0, C)], u_b.at[b], sems[b]).wait()
        pltpu.make_async_copy(ht.at[pl.ds(0, C)], t_b.at[b], sems[b]).wait()

    lane = jnp.arange(16, dtype=jnp.int32)

    def compute(g, b):
        u2 = u_b.at[b]
        t2 = t_b.at[b]
        for gi in range(C // 16):
            rows = gi * 16 + lane

            def jbody(jj, acc):
                for k in range(UNROLL):
                    cols = jnp.full((16,), jj * UNROLL + k, dtype=jnp.int32)
                    uu = plsc.load_gather(u2, [rows, cols])
                    tt = plsc.load_gather(t2, [rows, cols])
                    acc = acc + uu * tt
                return acc

            acc = lax.fori_loop(0, D // UNROLL, jbody,
                                jnp.zeros((16,), jnp.float32))
            out_v[g, pl.ds(gi * 16, 16)] = acc

    for b in range(NBUF):
        fire(b, b)

    def outer(i, carry):
        g0 = i * NBUF
        for b in range(NBUF):
            g = g0 + b
            wait(b)
            compute(g, b)

            @pl.when(g + NBUF < NCHUNK)
            def _():
                fire(g + NBUF, b)
        return carry

    lax.fori_loop(0, NCHUNK // NBUF, outer, 0)

    pltpu.sync_copy(out_v, out_hbm.at[wid])


def kernel(h_user, h_track, edge_index):
    src = edge_index[0].astype(jnp.int32)
    dst = edge_index[1].astype(jnp.int32)
    pad = EP - E
    src = jnp.concatenate([src, jnp.zeros((pad,), jnp.int32)])
    dst = jnp.concatenate([dst, jnp.zeros((pad,), jnp.int32)])
    out = _dot_edges(h_user, h_track,
                     src.reshape(NW, NCHUNK, C), dst.reshape(NW, NCHUNK, C))
    return out.reshape(EP)[:E]


# speedup vs baseline: 3.1944x; 3.1944x over previous
import functools

import jax
import jax.numpy as jnp
from jax import lax
from jax.experimental import pallas as pl
from jax.experimental.pallas import tpu as pltpu
from jax.experimental.pallas import tpu_sc as plsc

E = 160000
D = 256
NC = 2
NS = 16
NW = NC * NS
EP = 163840
EW = EP // NW
C = 64
NCHUNK = EW // C
NBUF = 2
UNROLL = 8

_mesh = plsc.VectorSubcoreMesh(core_axis_name="c", subcore_axis_name="s")


@functools.partial(
    pl.kernel,
    mesh=_mesh,
    compiler_params=pltpu.CompilerParams(use_tc_tiling_on_sc=False,
                                         needs_layout_passes=False),
    out_type=jax.ShapeDtypeStruct((NW, NCHUNK, C), jnp.float32),
    scratch_types=[
        pltpu.VMEM((NCHUNK, C), jnp.int32),
        pltpu.VMEM((NCHUNK, C), jnp.int32),
        pltpu.VMEM((NBUF, C, D), jnp.float32),
        pltpu.VMEM((NBUF, C, D), jnp.float32),
        pltpu.VMEM((NCHUNK, C), jnp.float32),
        pltpu.SemaphoreType.DMA,
        pltpu.SemaphoreType.DMA,
    ],
)
def _dot_edges(hu, ht, src_hbm, dst_hbm, out_hbm,
               src_v, dst_v, u_b, t_b, out_v, sem0, sem1):
    wid = lax.axis_index("s") * NC + lax.axis_index("c")
    sems = (sem0, sem1)

    pltpu.sync_copy(src_hbm.at[wid], src_v)
    pltpu.sync_copy(dst_hbm.at[wid], dst_v)

    def fire(g, b):
        pltpu.async_copy(hu.at[src_v.at[g]], u_b.at[b], sems[b])
        pltpu.async_copy(ht.at[dst_v.at[g]], t_b.at[b], sems[b])

    def wait(b):
        pltpu.make_async_copy(hu.at[pl.ds(0, C)], u_b.at[b], sems[b]).wait()
        pltpu.make_async_copy(ht.at[pl.ds(0, C)], t_b.at[b], sems[b]).wait()

    lane = jnp.arange(16, dtype=jnp.int32)

    def compute(g, b):
        u2 = u_b.at[b]
        t2 = t_b.at[b]

        def group_body(gi, carry):
            res = jnp.zeros((16,), jnp.float32)
            for ei in range(16):
                e = gi * 16 + ei
                ps = [u2[e, pl.ds(k * 16, 16)] * t2[e, pl.ds(k * 16, 16)]
                      for k in range(D // 16)]
                while len(ps) > 1:
                    ps = [ps[i] + ps[i + 1] for i in range(0, len(ps), 2)]
                s = jnp.sum(ps[0])
                res = jnp.where(lane == ei, s, res)
            out_v[g, pl.ds(gi * 16, 16)] = res
            return carry

        lax.fori_loop(0, C // 16, group_body, 0)

    for b in range(NBUF):
        fire(b, b)

    def outer(i, carry):
        g0 = i * NBUF
        for b in range(NBUF):
            g = g0 + b
            wait(b)
            compute(g, b)

            @pl.when(g + NBUF < NCHUNK)
            def _():
                fire(g + NBUF, b)
        return carry

    lax.fori_loop(0, NCHUNK // NBUF, outer, 0)

    pltpu.sync_copy(out_v, out_hbm.at[wid])


def kernel(h_user, h_track, edge_index):
    src = edge_index[0].astype(jnp.int32)
    dst = edge_index[1].astype(jnp.int32)
    pad = EP - E
    src = jnp.concatenate([src, jnp.zeros((pad,), jnp.int32)])
    dst = jnp.concatenate([dst, jnp.zeros((pad,), jnp.int32)])
    out = _dot_edges(h_user, h_track,
                     src.reshape(NW, NCHUNK, C), dst.reshape(NW, NCHUNK, C))
    return out.reshape(EP)[:E]
